# Initial kernel scaffold; baseline (speedup 1.0000x reference)
import jax
import jax.numpy as jnp
from jax.experimental import pallas as pl


def kernel(edge_index, edge_label_index, emb, lin1_w, lin2_w):
    raise NotImplementedError("write your pallas kernel here")


# speedup vs baseline: 8.4954x; 8.4954x over previous
import functools

import jax
import jax.numpy as jnp
from jax import lax
from jax.experimental import pallas as pl
from jax.experimental.pallas import tpu as pltpu
from jax.experimental.pallas import tpu_sc as plsc

N = 50000
D = 64
E = 800000
NLAB = 16384
NC = 2
NS = 16
LANES = 16
CH = 128

E_PAD = 802816
CHUNKS_A = 196
CHUNKS_D = 392
IB = 14
NGRP = CHUNKS_D // IB

HALF = 25024
TRASH = 25024
AGG_ROWS = 25088
ZROWS = 1568
WT = 1568
W0_REM = HALF - 15 * WT
W1_REM = (N - HALF) - 15 * WT

DEGW_ROWS = 51200
DEG_OUT = 50048

_MESH = dict(core_axis_name="c", subcore_axis_name="s")
_SC_PARAMS = dict(
    compiler_params=pltpu.CompilerParams(use_tc_tiling_on_sc=False))


def _deg_kernel(col_a):
    mesh = plsc.VectorSubcoreMesh(**_MESH)

    @functools.partial(
        pl.kernel,
        out_type=jax.ShapeDtypeStruct((NC, DEG_OUT, LANES), jnp.float32),
        mesh=mesh,
        scratch_types=[
            pltpu.VMEM((CHUNKS_A, CH), jnp.int32),
            pltpu.VMEM((CH, LANES), jnp.float32),
            pltpu.VMEM_SHARED((DEGW_ROWS, LANES), jnp.float32),
        ],
        **_SC_PARAMS,
    )
    def k(col_hbm, out_hbm, idx_v, ones_v, degw):
        c = lax.axis_index("c")
        s = lax.axis_index("s")
        wid = c * NS + s

        def fill_zero(i, carry):
            ones_v[i] = jnp.zeros((LANES,), jnp.float32)
            return carry

        lax.fori_loop(0, CH, fill_zero, 0)
        zbase = s * (DEGW_ROWS // NS)

        def zfill(kk, carry):
            pltpu.sync_copy(ones_v, degw.at[pl.ds(zbase + kk * CH, CH)])
            return carry

        lax.fori_loop(0, DEGW_ROWS // NS // CH, zfill, 0)

        def fill_one(i, carry):
            ones_v[i] = jnp.ones((LANES,), jnp.float32)
            return carry

        lax.fori_loop(0, CH, fill_one, 0)
        pltpu.sync_copy(col_hbm.at[wid], idx_v)
        plsc.subcore_barrier()

        def scat(j, carry):
            pltpu.sync_copy(ones_v, degw.at[idx_v.at[j]], add=True)
            return carry

        lax.fori_loop(0, CHUNKS_A, scat, 0)
        plsc.subcore_barrier()
        obase = s * (DEG_OUT // NS)
        pltpu.sync_copy(degw.at[pl.ds(obase, DEG_OUT // NS)],
                        out_hbm.at[c, pl.ds(obase, DEG_OUT // NS)])

    return k(col_a)


def _prep_kernel(degw, row_p):
    def dis_body(degw_ref, dis_ref):
        d = degw_ref[0] + degw_ref[1]
        dis_ref[...] = jnp.where(d > 0.5, lax.rsqrt(jnp.maximum(d, 1.0)), 0.0)

    dis = pl.pallas_call(
        dis_body,
        grid=(N // BR,),
        in_specs=[pl.BlockSpec((2, BR, LANES), lambda i: (0, i, 0))],
        out_specs=pl.BlockSpec((BR, LANES), lambda i: (i, 0)),
        out_shape=jax.ShapeDtypeStruct((N, LANES), jnp.float32),
    )(degw)

    def rl_body(row_ref, rl_ref):
        r = row_ref[...]
        rl0 = jnp.where((r >= 0) & (r < HALF), r, TRASH)
        rl1 = jnp.where((r >= HALF) & (r < N), r - HALF, TRASH)
        rl_ref[...] = jnp.stack([rl0, rl1])

    nrow = E_PAD // CH
    rl = pl.pallas_call(
        rl_body,
        grid=(nrow // CH,),
        in_specs=[pl.BlockSpec((CH, CH), lambda i: (i, 0))],
        out_specs=pl.BlockSpec((2, CH, CH), lambda i: (0, i, 0)),
        out_shape=jax.ShapeDtypeStruct((2, nrow, CH), jnp.int32),
    )(row_p)
    return dis, rl


def _agg_kernel(msg, col_d, rl):
    mesh = plsc.VectorSubcoreMesh(**_MESH)

    @functools.partial(
        pl.kernel,
        out_type=jax.ShapeDtypeStruct((N, D), jnp.float32),
        mesh=mesh,
        scratch_types=[
            pltpu.VMEM((IB, CH), jnp.int32),
            pltpu.VMEM((IB, CH), jnp.int32),
            pltpu.VMEM((IB, CH), jnp.int32),
            pltpu.VMEM((IB, CH), jnp.int32),
            pltpu.VMEM((CH, D), jnp.float32),
            pltpu.VMEM((CH, D), jnp.float32),
            pltpu.VMEM_SHARED((AGG_ROWS, D), jnp.float32),
            pltpu.SemaphoreType.DMA,
            pltpu.SemaphoreType.DMA,
            pltpu.SemaphoreType.DMA,
            pltpu.SemaphoreType.DMA,
            pltpu.SemaphoreType.DMA,
            pltpu.SemaphoreType.DMA,
        ],
        **_SC_PARAMS,
    )
    def k(msg_hbm, col_hbm, rl_hbm, out_hbm,
          cb0, cb1, rb0, rb1, g0, g1, agg,
          sc0, sc1, sr0, sr1, sg0, sg1):
        c = lax.axis_index("c")
        s = lax.axis_index("s")
        cb = (cb0, cb1)
        rb = (rb0, rb1)
        gb = (g0, g1)
        scs = (sc0, sc1)
        srs = (sr0, sr1)
        sgs = (sg0, sg1)

        def zrow(i, carry):
            for kk in range(D // LANES):
                g0[i, pl.ds(kk * LANES, LANES)] = jnp.zeros((LANES,), jnp.float32)
            return carry

        lax.fori_loop(0, CH, zrow, 0)
        zbase = s * ZROWS

        def zfill(kk, carry):
            pltpu.sync_copy(g0, agg.at[pl.ds(zbase + kk * CH, CH)])
            return carry

        lax.fori_loop(0, ZROWS // CH, zfill, 0)
        rem = ZROWS - (ZROWS // CH) * CH
        if rem:
            pltpu.sync_copy(g0.at[pl.ds(0, rem)],
                            agg.at[pl.ds(zbase + (ZROWS // CH) * CH, rem)])
        plsc.subcore_barrier()

        def issue_block(g, bb):
            pltpu.async_copy(col_hbm.at[s, pl.ds(g * IB, IB)], cb[bb], scs[bb])
            pltpu.async_copy(rl_hbm.at[c, s, pl.ds(g * IB, IB)], rb[bb], srs[bb])

        def do_group(g, bb):
            pltpu.make_async_copy(col_hbm.at[s, pl.ds(g * IB, IB)],
                                  cb[bb], scs[bb]).wait()
            pltpu.make_async_copy(rl_hbm.at[c, s, pl.ds(g * IB, IB)],
                                  rb[bb], srs[bb]).wait()
            pltpu.async_copy(msg_hbm.at[cb[bb].at[0]], gb[0], sgs[0])
            pltpu.async_copy(msg_hbm.at[cb[bb].at[1]], gb[1], sgs[1])
            for j in range(IB):
                p = j % 2
                pltpu.make_async_copy(msg_hbm.at[cb[bb].at[j]], gb[p],
                                      sgs[p]).wait()
                pltpu.sync_copy(gb[p], agg.at[rb[bb].at[j]], add=True)
                if j + 2 < IB:
                    pltpu.async_copy(msg_hbm.at[cb[bb].at[j + 2]], gb[p],
                                     sgs[p])
            @pl.when(g + 2 < NGRP)
            def _():
                issue_block(g + 2, bb)

        issue_block(0, 0)
        issue_block(1, 1)

        def body(gp, carry):
            do_group(2 * gp, 0)
            do_group(2 * gp + 1, 1)
            return carry

        lax.fori_loop(0, NGRP // 2, body, 0)
        plsc.subcore_barrier()

        @pl.when((c == 0) & (s < 15))
        def _():
            pltpu.sync_copy(agg.at[pl.ds(s * WT, WT)],
                            out_hbm.at[pl.ds(s * WT, WT)])

        @pl.when((c == 0) & (s == 15))
        def _():
            pltpu.sync_copy(agg.at[pl.ds(15 * WT, W0_REM)],
                            out_hbm.at[pl.ds(15 * WT, W0_REM)])

        @pl.when((c == 1) & (s < 15))
        def _():
            pltpu.sync_copy(agg.at[pl.ds(s * WT, WT)],
                            out_hbm.at[pl.ds(HALF + s * WT, WT)])

        @pl.when((c == 1) & (s == 15))
        def _():
            pltpu.sync_copy(agg.at[pl.ds(15 * WT, W1_REM)],
                            out_hbm.at[pl.ds(HALF + 15 * WT, W1_REM)])

    return k(msg, col_d, rl)


BR = 400
_DN = (((1,), (1,)), ((), ()))


def _mm_first(x, dis, w1, w2):
    def body(x_ref, dis_ref, w1_ref, w2_ref, msg_ref):
        xb = x_ref[...]
        m = (lax.dot_general(xb, w1_ref[...], _DN, preferred_element_type=jnp.float32)
             + lax.dot_general(xb * xb, w2_ref[...], _DN, preferred_element_type=jnp.float32))
        msg_ref[...] = m * dis_ref[...][:, 0:1]

    return pl.pallas_call(
        body,
        grid=(N // BR,),
        in_specs=[pl.BlockSpec((BR, D), lambda i: (i, 0)),
                  pl.BlockSpec((BR, LANES), lambda i: (i, 0)),
                  pl.BlockSpec((D, D), lambda i: (0, 0)),
                  pl.BlockSpec((D, D), lambda i: (0, 0))],
        out_specs=pl.BlockSpec((BR, D), lambda i: (i, 0)),
        out_shape=jax.ShapeDtypeStruct((N, D), jnp.float32),
    )(x, dis, w1, w2)


def _mm_mid(agg, dis, w1, w2):
    def body(a_ref, dis_ref, w1_ref, w2_ref, x_ref, msg_ref):
        dv = dis_ref[...][:, 0:1]
        a = a_ref[...] * dv
        xb = jnp.where(a >= 0, a, 0.2 * a)
        x_ref[...] = xb
        m = (lax.dot_general(xb, w1_ref[...], _DN, preferred_element_type=jnp.float32)
             + lax.dot_general(xb * xb, w2_ref[...], _DN, preferred_element_type=jnp.float32))
        msg_ref[...] = m * dv

    return pl.pallas_call(
        body,
        grid=(N // BR,),
        in_specs=[pl.BlockSpec((BR, D), lambda i: (i, 0)),
                  pl.BlockSpec((BR, LANES), lambda i: (i, 0)),
                  pl.BlockSpec((D, D), lambda i: (0, 0)),
                  pl.BlockSpec((D, D), lambda i: (0, 0))],
        out_specs=[pl.BlockSpec((BR, D), lambda i: (i, 0)),
                   pl.BlockSpec((BR, D), lambda i: (i, 0))],
        out_shape=(jax.ShapeDtypeStruct((N, D), jnp.float32),
                   jax.ShapeDtypeStruct((N, D), jnp.float32)),
    )(agg, dis, w1, w2)


def _leaky_kernel(agg, dis):
    def body(a_ref, dis_ref, x_ref):
        a = a_ref[...] * dis_ref[...][:, 0:1]
        x_ref[...] = jnp.where(a >= 0, a, 0.2 * a)

    return pl.pallas_call(
        body,
        grid=(N // BR,),
        in_specs=[pl.BlockSpec((BR, D), lambda i: (i, 0)),
                  pl.BlockSpec((BR, LANES), lambda i: (i, 0))],
        out_specs=pl.BlockSpec((BR, D), lambda i: (i, 0)),
        out_shape=jax.ShapeDtypeStruct((N, D), jnp.float32),
    )(agg, dis)


def _score_gather(emb, x1, x2, x3, eli):
    mesh = plsc.VectorSubcoreMesh(**_MESH)
    LCH = NLAB // (NC * NS) // CH

    @functools.partial(
        pl.kernel,
        out_type=(jax.ShapeDtypeStruct((4, NLAB, D), jnp.float32),
                  jax.ShapeDtypeStruct((4, NLAB, D), jnp.float32)),
        mesh=mesh,
        scratch_types=[
            pltpu.VMEM((LCH, CH), jnp.int32),
            pltpu.VMEM((LCH, CH), jnp.int32),
            pltpu.VMEM((CH, D), jnp.float32),
            pltpu.VMEM((CH, D), jnp.float32),
            pltpu.SemaphoreType.DMA,
            pltpu.SemaphoreType.DMA,
        ],
        **_SC_PARAMS,
    )
    def k(emb_h, x1_h, x2_h, x3_h, eli_h, src_out, dst_out,
          sidx, didx, g0, g1, s0, s1):
        c = lax.axis_index("c")
        s = lax.axis_index("s")
        wid = c * NS + s
        pltpu.sync_copy(eli_h.at[0, wid], sidx)
        pltpu.sync_copy(eli_h.at[1, wid], didx)
        tables = (emb_h, x1_h, x2_h, x3_h)
        tasks = []
        for side in range(2):
            idx = sidx if side == 0 else didx
            out = src_out if side == 0 else dst_out
            for tk in range(4):
                for j in range(LCH):
                    tasks.append((tables[tk], idx, j, out, tk))
        bufs = (g0, g1)
        sems = (s0, s1)

        def issue(i, b):
            tab, idx, j, _, _ = tasks[i]
            pltpu.async_copy(tab.at[idx.at[j]], bufs[b], sems[b])

        def drain(i, b):
            tab, idx, j, out, tk = tasks[i]
            pltpu.make_async_copy(tab.at[idx.at[j]], bufs[b], sems[b]).wait()
            pltpu.sync_copy(bufs[b],
                            out.at[tk, pl.ds(wid * (LCH * CH) + j * CH, CH)])

        nt = len(tasks)
        issue(0, 0)
        issue(1, 1)
        for i in range(2, nt):
            drain(i - 2, i % 2)
            issue(i, i % 2)
        drain(nt - 2, 0)
        drain(nt - 1, 1)

    return k(emb, x1, x2, x3, eli)


def _score_dot(src_g, dst_g):
    SB = 128

    def body(a_ref, b_ref, o_ref):
        p = a_ref[...] * b_ref[...]
        o_ref[...] = jnp.sum(p, axis=(0, 2))[:, None]

    return pl.pallas_call(
        body,
        grid=(NLAB // SB,),
        in_specs=[pl.BlockSpec((4, SB, D), lambda i: (0, i, 0)),
                  pl.BlockSpec((4, SB, D), lambda i: (0, i, 0))],
        out_specs=pl.BlockSpec((SB, 1), lambda i: (i, 0)),
        out_shape=jax.ShapeDtypeStruct((NLAB, 1), jnp.float32),
    )(src_g, dst_g)


def kernel(edge_index, edge_label_index, emb, lin1_w, lin2_w):
    row = edge_index[0]
    col = edge_index[1]
    pad = E_PAD - E
    col_a = jnp.pad(col, (0, pad), constant_values=N).reshape(NC * NS, CHUNKS_A, CH)
    col_d = jnp.pad(col, (0, pad)).reshape(NS, CHUNKS_D, CH)
    row_p = jnp.pad(row, (0, pad), constant_values=-1).reshape(E_PAD // CH, CH)
    eli = edge_label_index.reshape(2, NC * NS, NLAB // (NC * NS) // CH, CH)

    degw = _deg_kernel(col_a)
    dis, rl = _prep_kernel(degw, row_p)
    rl = rl.reshape(NC, NS, CHUNKS_D, CH)

    msg = _mm_first(emb, dis, lin1_w[0], lin2_w[0])
    agg = _agg_kernel(msg, col_d, rl)
    x1, msg = _mm_mid(agg, dis, lin1_w[1], lin2_w[1])
    agg = _agg_kernel(msg, col_d, rl)
    x2, msg = _mm_mid(agg, dis, lin1_w[2], lin2_w[2])
    agg = _agg_kernel(msg, col_d, rl)
    x3 = _leaky_kernel(agg, dis)

    src_g, dst_g = _score_gather(emb, x1, x2, x3, eli)
    scores = _score_dot(src_g, dst_g)
    return scores.reshape(NLAB)
